# baseline (speedup 1.0000x reference)
import functools

import jax
import jax.numpy as jnp
from jax import lax
from jax.experimental import pallas as pl
from jax.experimental.pallas import tpu as pltpu
from jax.experimental.pallas import tpu_sc as plsc

N = 2048
D = 768
DH = D // 2
HD = 512
E = 8
K = 2
BR = 256
PT = N * K + E * BR
NB = PT // BR
NBPAD = 64
CF = 1.25
ALPHA = 0.01

NW = 32
TPW = N // NW


def _pack(a, b):
    au = lax.bitcast_convert_type(a.astype(jnp.bfloat16), jnp.uint16)
    bu = lax.bitcast_convert_type(b.astype(jnp.bfloat16), jnp.uint16)
    word = au.astype(jnp.uint32) | (bu.astype(jnp.uint32) << 16)
    return lax.bitcast_convert_type(word, jnp.int32)


def _unpack(w):
    wu = lax.bitcast_convert_type(w, jnp.uint32)
    a = lax.bitcast_convert_type(wu << 16, jnp.float32)
    b = lax.bitcast_convert_type(wu & jnp.uint32(0xFFFF0000), jnp.float32)
    return a, b


def _router_body(x_ref, wr_ref, br_ref, w_out, x16_out, dest_out, smap_out,
                 aux_out):
    xf = x_ref[0]
    x16_out[...] = _pack(xf[:, :DH], xf[:, DH:])
    logits = jnp.dot(xf, wr_ref[...],
                     preferred_element_type=jnp.float32) + br_ref[...]
    m = jnp.max(logits, axis=1, keepdims=True)
    ex = jnp.exp(logits - m)
    sm = ex / jnp.sum(ex, axis=1, keepdims=True)

    iota = lax.broadcasted_iota(jnp.int32, (N, E), 1).astype(jnp.float32)
    m0 = jnp.max(sm, axis=1, keepdims=True)
    i0 = jnp.min(jnp.where(sm == m0, iota, jnp.float32(E)), axis=1,
                 keepdims=True)
    sm1 = jnp.where(iota == i0, -jnp.inf, sm)
    m1 = jnp.max(sm1, axis=1, keepdims=True)
    i1 = jnp.min(jnp.where(sm1 == m1, iota, jnp.float32(E)), axis=1,
                 keepdims=True)
    denom = m0 + m1 + 1e-9
    w_out[...] = jnp.concatenate([m0 / denom, m1 / denom], axis=1)

    oh0 = (iota == i0).astype(jnp.float32)
    oh1 = (iota == i1).astype(jnp.float32)
    oh = oh0 + oh1

    cnt = jnp.sum(oh, axis=0, keepdims=True)
    cap = CF * (N * K) / E
    aux_out[0, 0] = ALPHA * jnp.sum(jnp.maximum(cnt - cap, 0.0)) / E / N

    c = oh
    d = 1
    while d < N:
        c = c + jnp.concatenate(
            [jnp.zeros((d, E), jnp.float32), c[: N - d, :]], axis=0)
        d *= 2
    cex = c - oh

    p_e = jnp.ceil(cnt / BR) * BR
    s = p_e
    d = 1
    while d < E:
        s = s + jnp.concatenate(
            [jnp.zeros((1, d), jnp.float32), s[:, : E - d]], axis=1)
        d *= 2
    start_ex = s - p_e

    dest0 = (jnp.sum(oh0 * start_ex, axis=1, keepdims=True)
             + jnp.sum(oh0 * cex, axis=1, keepdims=True))
    dest1 = (jnp.sum(oh1 * start_ex, axis=1, keepdims=True)
             + jnp.sum(oh1 * cex, axis=1, keepdims=True))
    dest_out[...] = jnp.concatenate([dest0, dest1], axis=1).astype(jnp.int32)

    nb_e = p_e / BR
    cum_nb = s / BR
    nb_tot = jnp.sum(nb_e)
    biota = lax.broadcasted_iota(jnp.int32, (NBPAD, E), 0).astype(jnp.float32)
    be = jnp.sum((cum_nb <= biota).astype(jnp.float32), axis=1, keepdims=True)
    eidx = lax.broadcasted_iota(jnp.int32, (1, E), 1).astype(jnp.float32)
    last_e = jnp.max(jnp.where(cnt > 0, eidx, 0.0))
    em = jnp.minimum(be, last_e)
    bcol = lax.broadcasted_iota(jnp.int32, (NBPAD, 1), 0).astype(jnp.float32)
    act = (bcol < nb_tot).astype(jnp.float32)

    eidx_row = lax.broadcasted_iota(jnp.int32, (NBPAD, E), 1).astype(
        jnp.float32)
    active_row = (cnt > 0).astype(jnp.float32)
    ohb = (eidx_row == em).astype(jnp.float32)
    ps = jnp.sum(ohb * start_ex, axis=1, keepdims=True)
    first = ((bcol * BR) == ps).astype(jnp.float32) * act
    seg = jnp.sum(active_row * (eidx_row < em).astype(jnp.float32),
                  axis=1, keepdims=True)
    slot = seg - 2.0 * jnp.floor(seg / 2.0)
    nxtmat = jnp.where((active_row > 0) & (eidx_row > em), eidx_row, 99.0)
    nxt = jnp.min(nxtmat, axis=1, keepdims=True)
    hn = (nxt < 99.0).astype(jnp.float32) * act
    nxt = jnp.minimum(nxt, last_e)
    pad = jnp.zeros((NBPAD, 2), jnp.float32)
    smap_out[...] = jnp.concatenate(
        [em, act, first, slot, nxt, hn, pad], axis=1).astype(jnp.int32)


_router_call = pl.pallas_call(
    _router_body,
    out_shape=(
        jax.ShapeDtypeStruct((N, K), jnp.float32),
        jax.ShapeDtypeStruct((N, DH), jnp.int32),
        jax.ShapeDtypeStruct((N, K), jnp.int32),
        jax.ShapeDtypeStruct((NBPAD, 8), jnp.int32),
        jax.ShapeDtypeStruct((1, 1), jnp.float32),
    ),
    out_specs=(
        pl.BlockSpec(memory_space=pltpu.VMEM),
        pl.BlockSpec(memory_space=pltpu.VMEM),
        pl.BlockSpec(memory_space=pltpu.VMEM),
        pl.BlockSpec(memory_space=pltpu.VMEM),
        pl.BlockSpec(memory_space=pltpu.SMEM),
    ),
)


@functools.cache
def _sc_kernels():
    mesh = plsc.VectorSubcoreMesh(core_axis_name="c", subcore_axis_name="s")

    @functools.partial(
        pl.kernel,
        mesh=mesh,
        out_type=jax.ShapeDtypeStruct((PT, DH), jnp.int32),
        scratch_types=[
            pltpu.VMEM((TPW, DH), jnp.int32),
            pltpu.VMEM((TPW,), jnp.int32),
            pltpu.VMEM((TPW,), jnp.int32),
            pltpu.SemaphoreType.DMA,
        ],
    )
    def _dispatch(xf_hbm, destT_hbm, xs_hbm, xrows, idx0, idx1, sem):
        wid = lax.axis_index("s") * 2 + lax.axis_index("c")
        base = wid * TPW
        pltpu.sync_copy(xf_hbm.at[pl.ds(base, TPW)], xrows)
        pltpu.sync_copy(destT_hbm.at[0, pl.ds(base, TPW)], idx0)
        pltpu.sync_copy(destT_hbm.at[1, pl.ds(base, TPW)], idx1)
        pltpu.async_copy(xrows, xs_hbm.at[idx0], sem).wait()
        pltpu.async_copy(xrows, xs_hbm.at[idx1], sem).wait()

    @functools.partial(
        pl.kernel,
        mesh=mesh,
        out_type=(
            jax.ShapeDtypeStruct((N, DH), jnp.int32),
            jax.ShapeDtypeStruct((N, DH), jnp.int32),
        ),
        scratch_types=[
            pltpu.VMEM((TPW, DH), jnp.int32),
            pltpu.VMEM((TPW, DH), jnp.int32),
            pltpu.VMEM((TPW,), jnp.int32),
            pltpu.VMEM((TPW,), jnp.int32),
            pltpu.SemaphoreType.DMA,
        ],
    )
    def _combine_gather(ys_hbm, destT_hbm, ye_hbm, yo_hbm, arows, brows,
                        idx0, idx1, sem):
        wid = lax.axis_index("s") * 2 + lax.axis_index("c")
        base = wid * TPW
        pltpu.sync_copy(destT_hbm.at[0, pl.ds(base, TPW)], idx0)
        pltpu.sync_copy(destT_hbm.at[1, pl.ds(base, TPW)], idx1)
        pltpu.async_copy(ys_hbm.at[idx0], arows, sem).wait()
        pltpu.async_copy(ys_hbm.at[idx1], brows, sem).wait()
        pltpu.sync_copy(arows, ye_hbm.at[pl.ds(base, TPW)])
        pltpu.sync_copy(brows, yo_hbm.at[pl.ds(base, TPW)])

    return _dispatch, _combine_gather


def _ffn_body(smap_ref, xs_ref, wg_hbm, wu_hbm, wd_hbm, bg_ref, bu_ref,
              bd_ref, y_ref, wg_buf, wu_buf, wd_buf, wg16, wu16, wd16, sems):
    b = pl.program_id(0)
    s = smap_ref[b, 3]

    def _start(e, sl):
        pltpu.make_async_copy(wg_hbm.at[e], wg_buf.at[sl], sems.at[sl]).start()
        pltpu.make_async_copy(wu_hbm.at[e], wu_buf.at[sl], sems.at[sl]).start()
        pltpu.make_async_copy(wd_hbm.at[e], wd_buf.at[sl], sems.at[sl]).start()

    def _wait(e, sl):
        pltpu.make_async_copy(wg_hbm.at[e], wg_buf.at[sl], sems.at[sl]).wait()
        pltpu.make_async_copy(wu_hbm.at[e], wu_buf.at[sl], sems.at[sl]).wait()
        pltpu.make_async_copy(wd_hbm.at[e], wd_buf.at[sl], sems.at[sl]).wait()

    is_first = (smap_ref[b, 2] != 0) & (smap_ref[b, 1] != 0)

    @pl.when(b == 0)
    def _():
        _start(smap_ref[b, 0], s)

    @pl.when(is_first)
    def _():
        _wait(smap_ref[b, 0], s)
        wg16[...] = wg_buf[s].astype(jnp.bfloat16)
        wu16[...] = wu_buf[s].astype(jnp.bfloat16)
        wd16[...] = wd_buf[s].astype(jnp.bfloat16)

    @pl.when(is_first & (smap_ref[b, 5] != 0))
    def _():
        _start(smap_ref[b, 4], 1 - s)

    @pl.when(smap_ref[b, 1] != 0)
    def _():
        xa, xb_ = _unpack(xs_ref[...])
        xa = xa.astype(jnp.bfloat16)
        xb_ = xb_.astype(jnp.bfloat16)
        g = (jnp.dot(xa, wg16[:DH, :], preferred_element_type=jnp.float32)
             + jnp.dot(xb_, wg16[DH:, :], preferred_element_type=jnp.float32)
             + bg_ref[0])
        u = (jnp.dot(xa, wu16[:DH, :], preferred_element_type=jnp.float32)
             + jnp.dot(xb_, wu16[DH:, :], preferred_element_type=jnp.float32)
             + bu_ref[0])
        h = (g * (u * jax.nn.sigmoid(u))).astype(jnp.bfloat16)
        ya = (jnp.dot(h, wd16[:, :DH], preferred_element_type=jnp.float32)
              + bd_ref[0][:, :DH])
        yb = (jnp.dot(h, wd16[:, DH:], preferred_element_type=jnp.float32)
              + bd_ref[0][:, DH:])
        y_ref[...] = _pack(ya, yb)


_ffn_call = pl.pallas_call(
    _ffn_body,
    grid_spec=pltpu.PrefetchScalarGridSpec(
        num_scalar_prefetch=1,
        grid=(NB,),
        in_specs=[
            pl.BlockSpec((BR, DH), lambda b, sm: (b, 0)),
            pl.BlockSpec(memory_space=pl.ANY),
            pl.BlockSpec(memory_space=pl.ANY),
            pl.BlockSpec(memory_space=pl.ANY),
            pl.BlockSpec((1, 1, HD), lambda b, sm: (sm[b, 0], 0, 0)),
            pl.BlockSpec((1, 1, HD), lambda b, sm: (sm[b, 0], 0, 0)),
            pl.BlockSpec((1, 1, D), lambda b, sm: (sm[b, 0], 0, 0)),
        ],
        out_specs=pl.BlockSpec((BR, DH), lambda b, sm: (b, 0)),
        scratch_shapes=[
            pltpu.VMEM((2, D, HD), jnp.float32),
            pltpu.VMEM((2, D, HD), jnp.float32),
            pltpu.VMEM((2, HD, D), jnp.float32),
            pltpu.VMEM((D, HD), jnp.bfloat16),
            pltpu.VMEM((D, HD), jnp.bfloat16),
            pltpu.VMEM((HD, D), jnp.bfloat16),
            pltpu.SemaphoreType.DMA((2,)),
        ],
    ),
    out_shape=jax.ShapeDtypeStruct((PT, DH), jnp.int32),
)


def _combine_body(ye_ref, yo_ref, w_ref, out_ref):
    ea, eb = _unpack(ye_ref[...])
    oa, ob = _unpack(yo_ref[...])
    w0 = w_ref[:, 0:1]
    w1 = w_ref[:, 1:2]
    out_ref[0] = jnp.concatenate(
        [ea * w0 + oa * w1, eb * w0 + ob * w1], axis=1)


_combine_call = pl.pallas_call(
    _combine_body,
    grid=(N // 512,),
    in_specs=[
        pl.BlockSpec((512, DH), lambda b: (b, 0)),
        pl.BlockSpec((512, DH), lambda b: (b, 0)),
        pl.BlockSpec((512, K), lambda b: (b, 0)),
    ],
    out_specs=pl.BlockSpec((1, 512, D), lambda b: (0, b, 0)),
    out_shape=jax.ShapeDtypeStruct((1, N, D), jnp.float32),
)


def kernel(x, Wr, br, Wg, bg, Wu, bu, Wd, bd):
    Bv, Sv, Dv = x.shape
    (w, x16, dest, smap, aux) = _router_call(x, Wr, br.reshape(1, E))
    destT = dest.T
    dispatch, combine_gather = _sc_kernels()
    xs = dispatch(x16, destT)
    ys = _ffn_call(smap, xs, Wg, Wu, Wd, bg.reshape(E, 1, HD),
                   bu.reshape(E, 1, HD), bd.reshape(E, 1, D))
    ye, yo = combine_gather(ys, destT)
    out = _combine_call(ye, yo, w)
    return out, aux[0, 0]


# speedup vs baseline: 1.0277x; 1.0277x over previous
import functools

import jax
import jax.numpy as jnp
from jax import lax
from jax.experimental import pallas as pl
from jax.experimental.pallas import tpu as pltpu
from jax.experimental.pallas import tpu_sc as plsc

N = 2048
D = 768
DH = D // 2
HD = 512
E = 8
K = 2
BR = 256
PT = N * K + E * BR
NB = PT // BR
NBPAD = 64
CF = 1.25
ALPHA = 0.01

NW = 32
TPW = N // NW


def _pack(a, b):
    au = lax.bitcast_convert_type(a.astype(jnp.bfloat16), jnp.uint16)
    bu = lax.bitcast_convert_type(b.astype(jnp.bfloat16), jnp.uint16)
    word = au.astype(jnp.uint32) | (bu.astype(jnp.uint32) << 16)
    return lax.bitcast_convert_type(word, jnp.int32)


def _unpack(w):
    wu = lax.bitcast_convert_type(w, jnp.uint32)
    a = lax.bitcast_convert_type(wu << 16, jnp.float32)
    b = lax.bitcast_convert_type(wu & jnp.uint32(0xFFFF0000), jnp.float32)
    return a, b


def _router_body(x_ref, wr_ref, br_ref, w_out, x16_out, dest_out, smap_out,
                 aux_out):
    xf = x_ref[0]
    x16_out[...] = _pack(xf[:, :DH], xf[:, DH:])
    logits = jnp.dot(xf, wr_ref[...],
                     preferred_element_type=jnp.float32) + br_ref[...]
    m = jnp.max(logits, axis=1, keepdims=True)
    ex = jnp.exp(logits - m)
    sm = ex / jnp.sum(ex, axis=1, keepdims=True)

    iota = lax.broadcasted_iota(jnp.int32, (N, E), 1).astype(jnp.float32)
    m0 = jnp.max(sm, axis=1, keepdims=True)
    i0 = jnp.min(jnp.where(sm == m0, iota, jnp.float32(E)), axis=1,
                 keepdims=True)
    sm1 = jnp.where(iota == i0, -jnp.inf, sm)
    m1 = jnp.max(sm1, axis=1, keepdims=True)
    i1 = jnp.min(jnp.where(sm1 == m1, iota, jnp.float32(E)), axis=1,
                 keepdims=True)
    denom = m0 + m1 + 1e-9
    w_out[...] = jnp.concatenate([m0 / denom, m1 / denom], axis=1)

    oh0 = (iota == i0).astype(jnp.float32)
    oh1 = (iota == i1).astype(jnp.float32)
    oh = oh0 + oh1

    cnt = jnp.sum(oh, axis=0, keepdims=True)
    cap = CF * (N * K) / E
    aux_out[0, 0] = ALPHA * jnp.sum(jnp.maximum(cnt - cap, 0.0)) / E / N

    c = oh
    d = 1
    while d < N:
        c = c + jnp.concatenate(
            [jnp.zeros((d, E), jnp.float32), c[: N - d, :]], axis=0)
        d *= 2
    cex = c - oh

    p_e = jnp.ceil(cnt / BR) * BR
    s = p_e
    d = 1
    while d < E:
        s = s + jnp.concatenate(
            [jnp.zeros((1, d), jnp.float32), s[:, : E - d]], axis=1)
        d *= 2
    start_ex = s - p_e

    dest0 = (jnp.sum(oh0 * start_ex, axis=1, keepdims=True)
             + jnp.sum(oh0 * cex, axis=1, keepdims=True))
    dest1 = (jnp.sum(oh1 * start_ex, axis=1, keepdims=True)
             + jnp.sum(oh1 * cex, axis=1, keepdims=True))
    dest_out[...] = jnp.transpose(
        jnp.concatenate([dest0, dest1], axis=1).astype(jnp.int32))

    nb_e = p_e / BR
    cum_nb = s / BR
    nb_tot = jnp.sum(nb_e)
    biota = lax.broadcasted_iota(jnp.int32, (NBPAD, E), 0).astype(jnp.float32)
    be = jnp.sum((cum_nb <= biota).astype(jnp.float32), axis=1, keepdims=True)
    eidx = lax.broadcasted_iota(jnp.int32, (1, E), 1).astype(jnp.float32)
    last_e = jnp.max(jnp.where(cnt > 0, eidx, 0.0))
    em = jnp.minimum(be, last_e)
    bcol = lax.broadcasted_iota(jnp.int32, (NBPAD, 1), 0).astype(jnp.float32)
    act = (bcol < nb_tot).astype(jnp.float32)

    eidx_row = lax.broadcasted_iota(jnp.int32, (NBPAD, E), 1).astype(
        jnp.float32)
    active_row = (cnt > 0).astype(jnp.float32)
    ohb = (eidx_row == em).astype(jnp.float32)
    ps = jnp.sum(ohb * start_ex, axis=1, keepdims=True)
    first = ((bcol * BR) == ps).astype(jnp.float32) * act
    seg = jnp.sum(active_row * (eidx_row < em).astype(jnp.float32),
                  axis=1, keepdims=True)
    slot = seg - 2.0 * jnp.floor(seg / 2.0)
    nxtmat = jnp.where((active_row > 0) & (eidx_row > em), eidx_row, 99.0)
    nxt = jnp.min(nxtmat, axis=1, keepdims=True)
    hn = (nxt < 99.0).astype(jnp.float32) * act
    nxt = jnp.minimum(nxt, last_e)
    pad = jnp.zeros((NBPAD, 2), jnp.float32)
    smap_out[...] = jnp.concatenate(
        [em, act, first, slot, nxt, hn, pad], axis=1).astype(jnp.int32)


_router_call = pl.pallas_call(
    _router_body,
    out_shape=(
        jax.ShapeDtypeStruct((N, K), jnp.float32),
        jax.ShapeDtypeStruct((N, DH), jnp.int32),
        jax.ShapeDtypeStruct((K, N), jnp.int32),
        jax.ShapeDtypeStruct((NBPAD, 8), jnp.int32),
        jax.ShapeDtypeStruct((1, 1), jnp.float32),
    ),
    out_specs=(
        pl.BlockSpec(memory_space=pltpu.VMEM),
        pl.BlockSpec(memory_space=pltpu.VMEM),
        pl.BlockSpec(memory_space=pltpu.VMEM),
        pl.BlockSpec(memory_space=pltpu.VMEM),
        pl.BlockSpec(memory_space=pltpu.SMEM),
    ),
)


@functools.cache
def _sc_kernels():
    mesh = plsc.VectorSubcoreMesh(core_axis_name="c", subcore_axis_name="s")

    @functools.partial(
        pl.kernel,
        mesh=mesh,
        out_type=jax.ShapeDtypeStruct((PT, DH), jnp.int32),
        scratch_types=[
            pltpu.VMEM((TPW, DH), jnp.int32),
            pltpu.VMEM((TPW,), jnp.int32),
            pltpu.VMEM((TPW,), jnp.int32),
            pltpu.SemaphoreType.DMA,
        ],
    )
    def _dispatch(xf_hbm, destT_hbm, xs_hbm, xrows, idx0, idx1, sem):
        wid = lax.axis_index("s") * 2 + lax.axis_index("c")
        base = wid * TPW
        pltpu.sync_copy(xf_hbm.at[pl.ds(base, TPW)], xrows)
        pltpu.sync_copy(destT_hbm.at[0, pl.ds(base, TPW)], idx0)
        pltpu.sync_copy(destT_hbm.at[1, pl.ds(base, TPW)], idx1)
        pltpu.async_copy(xrows, xs_hbm.at[idx0], sem).wait()
        pltpu.async_copy(xrows, xs_hbm.at[idx1], sem).wait()

    @functools.partial(
        pl.kernel,
        mesh=mesh,
        out_type=(
            jax.ShapeDtypeStruct((N, DH), jnp.int32),
            jax.ShapeDtypeStruct((N, DH), jnp.int32),
        ),
        scratch_types=[
            pltpu.VMEM((TPW, DH), jnp.int32),
            pltpu.VMEM((TPW, DH), jnp.int32),
            pltpu.VMEM((TPW,), jnp.int32),
            pltpu.VMEM((TPW,), jnp.int32),
            pltpu.SemaphoreType.DMA,
        ],
    )
    def _combine_gather(ys_hbm, destT_hbm, ye_hbm, yo_hbm, arows, brows,
                        idx0, idx1, sem):
        wid = lax.axis_index("s") * 2 + lax.axis_index("c")
        base = wid * TPW
        pltpu.sync_copy(destT_hbm.at[0, pl.ds(base, TPW)], idx0)
        pltpu.sync_copy(destT_hbm.at[1, pl.ds(base, TPW)], idx1)
        pltpu.async_copy(ys_hbm.at[idx0], arows, sem).wait()
        pltpu.async_copy(ys_hbm.at[idx1], brows, sem).wait()
        pltpu.sync_copy(arows, ye_hbm.at[pl.ds(base, TPW)])
        pltpu.sync_copy(brows, yo_hbm.at[pl.ds(base, TPW)])

    return _dispatch, _combine_gather


def _ffn_body(smap_ref, xs_ref, wg_hbm, wu_hbm, wd_hbm, bg_ref, bu_ref,
              bd_ref, y_ref, wg_buf, wu_buf, wd_buf, wg16, wu16, wd16, sems):
    b = pl.program_id(0)
    s = smap_ref[b, 3]

    def _start(e, sl):
        pltpu.make_async_copy(wg_hbm.at[e], wg_buf.at[sl], sems.at[sl]).start()
        pltpu.make_async_copy(wu_hbm.at[e], wu_buf.at[sl], sems.at[sl]).start()
        pltpu.make_async_copy(wd_hbm.at[e], wd_buf.at[sl], sems.at[sl]).start()

    def _wait(e, sl):
        pltpu.make_async_copy(wg_hbm.at[e], wg_buf.at[sl], sems.at[sl]).wait()
        pltpu.make_async_copy(wu_hbm.at[e], wu_buf.at[sl], sems.at[sl]).wait()
        pltpu.make_async_copy(wd_hbm.at[e], wd_buf.at[sl], sems.at[sl]).wait()

    is_first = (smap_ref[b, 2] != 0) & (smap_ref[b, 1] != 0)

    @pl.when(b == 0)
    def _():
        _start(smap_ref[b, 0], s)

    @pl.when(is_first)
    def _():
        _wait(smap_ref[b, 0], s)
        wg16[...] = wg_buf[s].astype(jnp.bfloat16)
        wu16[...] = wu_buf[s].astype(jnp.bfloat16)
        wd16[...] = wd_buf[s].astype(jnp.bfloat16)

    @pl.when(is_first & (smap_ref[b, 5] != 0))
    def _():
        _start(smap_ref[b, 4], 1 - s)

    @pl.when(smap_ref[b, 1] != 0)
    def _():
        xa, xb_ = _unpack(xs_ref[...])
        xa = xa.astype(jnp.bfloat16)
        xb_ = xb_.astype(jnp.bfloat16)
        g = (jnp.dot(xa, wg16[:DH, :], preferred_element_type=jnp.float32)
             + jnp.dot(xb_, wg16[DH:, :], preferred_element_type=jnp.float32)
             + bg_ref[0])
        u = (jnp.dot(xa, wu16[:DH, :], preferred_element_type=jnp.float32)
             + jnp.dot(xb_, wu16[DH:, :], preferred_element_type=jnp.float32)
             + bu_ref[0])
        h = (g * (u * jax.nn.sigmoid(u))).astype(jnp.bfloat16)
        ya = (jnp.dot(h, wd16[:, :DH], preferred_element_type=jnp.float32)
              + bd_ref[0][:, :DH])
        yb = (jnp.dot(h, wd16[:, DH:], preferred_element_type=jnp.float32)
              + bd_ref[0][:, DH:])
        y_ref[...] = _pack(ya, yb)


_ffn_call = pl.pallas_call(
    _ffn_body,
    grid_spec=pltpu.PrefetchScalarGridSpec(
        num_scalar_prefetch=1,
        grid=(NB,),
        in_specs=[
            pl.BlockSpec((BR, DH), lambda b, sm: (b, 0)),
            pl.BlockSpec(memory_space=pl.ANY),
            pl.BlockSpec(memory_space=pl.ANY),
            pl.BlockSpec(memory_space=pl.ANY),
            pl.BlockSpec((1, 1, HD), lambda b, sm: (sm[b, 0], 0, 0)),
            pl.BlockSpec((1, 1, HD), lambda b, sm: (sm[b, 0], 0, 0)),
            pl.BlockSpec((1, 1, D), lambda b, sm: (sm[b, 0], 0, 0)),
        ],
        out_specs=pl.BlockSpec((BR, DH), lambda b, sm: (b, 0)),
        scratch_shapes=[
            pltpu.VMEM((2, D, HD), jnp.float32),
            pltpu.VMEM((2, D, HD), jnp.float32),
            pltpu.VMEM((2, HD, D), jnp.float32),
            pltpu.VMEM((D, HD), jnp.bfloat16),
            pltpu.VMEM((D, HD), jnp.bfloat16),
            pltpu.VMEM((HD, D), jnp.bfloat16),
            pltpu.SemaphoreType.DMA((2,)),
        ],
    ),
    out_shape=jax.ShapeDtypeStruct((PT, DH), jnp.int32),
)


def _combine_body(ye_ref, yo_ref, w_ref, out_ref):
    ea, eb = _unpack(ye_ref[...])
    oa, ob = _unpack(yo_ref[...])
    w0 = w_ref[:, 0:1]
    w1 = w_ref[:, 1:2]
    out_ref[0] = jnp.concatenate(
        [ea * w0 + oa * w1, eb * w0 + ob * w1], axis=1)


_combine_call = pl.pallas_call(
    _combine_body,
    grid=(N // 512,),
    in_specs=[
        pl.BlockSpec((512, DH), lambda b: (b, 0)),
        pl.BlockSpec((512, DH), lambda b: (b, 0)),
        pl.BlockSpec((512, K), lambda b: (b, 0)),
    ],
    out_specs=pl.BlockSpec((1, 512, D), lambda b: (0, b, 0)),
    out_shape=jax.ShapeDtypeStruct((1, N, D), jnp.float32),
)


def kernel(x, Wr, br, Wg, bg, Wu, bu, Wd, bd):
    Bv, Sv, Dv = x.shape
    (w, x16, destT, smap, aux) = _router_call(x, Wr, br.reshape(1, E))
    dispatch, combine_gather = _sc_kernels()
    xs = dispatch(x16, destT)
    ys = _ffn_call(smap, xs, Wg, Wu, Wd, bg.reshape(E, 1, HD),
                   bu.reshape(E, 1, HD), bd.reshape(E, 1, D))
    ye, yo = combine_gather(ys, destT)
    out = _combine_call(ye, yo, w)
    return out, aux[0, 0]


# speedup vs baseline: 1.0434x; 1.0153x over previous
import functools

import jax
import jax.numpy as jnp
from jax import lax
from jax.experimental import pallas as pl
from jax.experimental.pallas import tpu as pltpu
from jax.experimental.pallas import tpu_sc as plsc

N = 2048
D = 768
DH = D // 2
HD = 512
E = 8
K = 2
BR = 256
PT = N * K + E * BR
NB = PT // BR
NBPAD = 64
CF = 1.25
ALPHA = 0.01

NW = 32
TPW = N // NW


def _pack(a, b):
    au = lax.bitcast_convert_type(a.astype(jnp.bfloat16), jnp.uint16)
    bu = lax.bitcast_convert_type(b.astype(jnp.bfloat16), jnp.uint16)
    word = au.astype(jnp.uint32) | (bu.astype(jnp.uint32) << 16)
    return lax.bitcast_convert_type(word, jnp.int32)


def _unpack(w):
    wu = lax.bitcast_convert_type(w, jnp.uint32)
    a = lax.bitcast_convert_type(wu << 16, jnp.float32)
    b = lax.bitcast_convert_type(wu & jnp.uint32(0xFFFF0000), jnp.float32)
    return a, b


def _router_body(x_ref, wr_ref, br_ref, w_out, x16_out, dest_out, smap_out,
                 aux_out):
    xf = x_ref[0]
    x16_out[...] = _pack(xf[:, :DH], xf[:, DH:])
    logits = jnp.dot(xf, wr_ref[...],
                     preferred_element_type=jnp.float32) + br_ref[...]
    m = jnp.max(logits, axis=1, keepdims=True)
    ex = jnp.exp(logits - m)
    sm = ex / jnp.sum(ex, axis=1, keepdims=True)

    iota = lax.broadcasted_iota(jnp.int32, (N, E), 1).astype(jnp.float32)
    m0 = jnp.max(sm, axis=1, keepdims=True)
    i0 = jnp.min(jnp.where(sm == m0, iota, jnp.float32(E)), axis=1,
                 keepdims=True)
    sm1 = jnp.where(iota == i0, -jnp.inf, sm)
    m1 = jnp.max(sm1, axis=1, keepdims=True)
    i1 = jnp.min(jnp.where(sm1 == m1, iota, jnp.float32(E)), axis=1,
                 keepdims=True)
    denom = m0 + m1 + 1e-9
    w_out[...] = jnp.concatenate([m0 / denom, m1 / denom], axis=1)

    oh0 = (iota == i0).astype(jnp.float32)
    oh1 = (iota == i1).astype(jnp.float32)
    oh = oh0 + oh1

    cnt = jnp.sum(oh, axis=0, keepdims=True)
    cap = CF * (N * K) / E
    aux_out[0, 0] = ALPHA * jnp.sum(jnp.maximum(cnt - cap, 0.0)) / E / N

    c = oh
    d = 1
    while d < N:
        c = c + jnp.concatenate(
            [jnp.zeros((d, E), jnp.float32), c[: N - d, :]], axis=0)
        d *= 2
    cex = c - oh

    p_e = jnp.ceil(cnt / BR) * BR
    s = p_e
    d = 1
    while d < E:
        s = s + jnp.concatenate(
            [jnp.zeros((1, d), jnp.float32), s[:, : E - d]], axis=1)
        d *= 2
    start_ex = s - p_e

    dest0 = (jnp.sum(oh0 * start_ex, axis=1, keepdims=True)
             + jnp.sum(oh0 * cex, axis=1, keepdims=True))
    dest1 = (jnp.sum(oh1 * start_ex, axis=1, keepdims=True)
             + jnp.sum(oh1 * cex, axis=1, keepdims=True))
    dest_out[...] = jnp.transpose(
        jnp.concatenate([dest0, dest1], axis=1).astype(jnp.int32))

    nb_e = p_e / BR
    cum_nb = s / BR
    nb_tot = jnp.sum(nb_e)
    biota = lax.broadcasted_iota(jnp.int32, (NBPAD, E), 0).astype(jnp.float32)
    be = jnp.sum((cum_nb <= biota).astype(jnp.float32), axis=1, keepdims=True)
    eidx = lax.broadcasted_iota(jnp.int32, (1, E), 1).astype(jnp.float32)
    last_e = jnp.max(jnp.where(cnt > 0, eidx, 0.0))
    em = jnp.minimum(be, last_e)
    bcol = lax.broadcasted_iota(jnp.int32, (NBPAD, 1), 0).astype(jnp.float32)
    act = (bcol < nb_tot).astype(jnp.float32)

    eidx_row = lax.broadcasted_iota(jnp.int32, (NBPAD, E), 1).astype(
        jnp.float32)
    active_row = (cnt > 0).astype(jnp.float32)
    ohb = (eidx_row == em).astype(jnp.float32)
    ps = jnp.sum(ohb * start_ex, axis=1, keepdims=True)
    first = ((bcol * BR) == ps).astype(jnp.float32) * act
    seg = jnp.sum(active_row * (eidx_row < em).astype(jnp.float32),
                  axis=1, keepdims=True)
    slot = seg - 2.0 * jnp.floor(seg / 2.0)
    nxtmat = jnp.where((active_row > 0) & (eidx_row > em), eidx_row, 99.0)
    nxt = jnp.min(nxtmat, axis=1, keepdims=True)
    hn = (nxt < 99.0).astype(jnp.float32) * act
    nxt = jnp.minimum(nxt, last_e)
    pad = jnp.zeros((NBPAD, 2), jnp.float32)
    smap_out[...] = jnp.concatenate(
        [em, act, first, slot, nxt, hn, pad], axis=1).astype(jnp.int32)


_router_call = pl.pallas_call(
    _router_body,
    out_shape=(
        jax.ShapeDtypeStruct((N, K), jnp.float32),
        jax.ShapeDtypeStruct((N, DH), jnp.int32),
        jax.ShapeDtypeStruct((K, N), jnp.int32),
        jax.ShapeDtypeStruct((NBPAD, 8), jnp.int32),
        jax.ShapeDtypeStruct((1, 1), jnp.float32),
    ),
    out_specs=(
        pl.BlockSpec(memory_space=pltpu.VMEM),
        pl.BlockSpec(memory_space=pltpu.VMEM),
        pl.BlockSpec(memory_space=pltpu.VMEM),
        pl.BlockSpec(memory_space=pltpu.VMEM),
        pl.BlockSpec(memory_space=pltpu.SMEM),
    ),
)


@functools.cache
def _sc_kernels():
    mesh = plsc.VectorSubcoreMesh(core_axis_name="c", subcore_axis_name="s")

    @functools.partial(
        pl.kernel,
        mesh=mesh,
        out_type=jax.ShapeDtypeStruct((PT, DH), jnp.int32),
        scratch_types=[
            pltpu.VMEM((TPW, DH), jnp.int32),
            pltpu.VMEM((TPW,), jnp.int32),
            pltpu.VMEM((TPW,), jnp.int32),
            pltpu.SemaphoreType.DMA,
        ],
    )
    def _dispatch(xf_hbm, destT_hbm, xs_hbm, xrows, idx0, idx1, sem):
        wid = lax.axis_index("s") * 2 + lax.axis_index("c")
        base = wid * TPW
        pltpu.sync_copy(xf_hbm.at[pl.ds(base, TPW)], xrows)
        pltpu.sync_copy(destT_hbm.at[0, pl.ds(base, TPW)], idx0)
        pltpu.sync_copy(destT_hbm.at[1, pl.ds(base, TPW)], idx1)
        pltpu.async_copy(xrows, xs_hbm.at[idx0], sem).wait()
        pltpu.async_copy(xrows, xs_hbm.at[idx1], sem).wait()

    @functools.partial(
        pl.kernel,
        mesh=mesh,
        out_type=(
            jax.ShapeDtypeStruct((N, DH), jnp.int32),
            jax.ShapeDtypeStruct((N, DH), jnp.int32),
        ),
        scratch_types=[
            pltpu.VMEM((TPW, DH), jnp.int32),
            pltpu.VMEM((TPW, DH), jnp.int32),
            pltpu.VMEM((TPW,), jnp.int32),
            pltpu.VMEM((TPW,), jnp.int32),
            pltpu.SemaphoreType.DMA,
        ],
    )
    def _combine_gather(ys_hbm, destT_hbm, ye_hbm, yo_hbm, arows, brows,
                        idx0, idx1, sem):
        wid = lax.axis_index("s") * 2 + lax.axis_index("c")
        base = wid * TPW
        pltpu.sync_copy(destT_hbm.at[0, pl.ds(base, TPW)], idx0)
        pltpu.sync_copy(destT_hbm.at[1, pl.ds(base, TPW)], idx1)
        pltpu.async_copy(ys_hbm.at[idx0], arows, sem).wait()
        pltpu.async_copy(ys_hbm.at[idx1], brows, sem).wait()
        pltpu.sync_copy(arows, ye_hbm.at[pl.ds(base, TPW)])
        pltpu.sync_copy(brows, yo_hbm.at[pl.ds(base, TPW)])

    return _dispatch, _combine_gather


def _ffn_body(smap_ref, xs_ref, wg_hbm, wu_hbm, wd_hbm, bg_ref, bu_ref,
              bd_ref, y_ref, wg_buf, wu_buf, wd_buf, wg16, wu16, wd16, sems):
    b = pl.program_id(0)
    s = smap_ref[b, 3]

    def _start(e, sl):
        pltpu.make_async_copy(wg_hbm.at[e], wg_buf.at[sl], sems.at[sl]).start()
        pltpu.make_async_copy(wu_hbm.at[e], wu_buf.at[sl], sems.at[sl]).start()
        pltpu.make_async_copy(wd_hbm.at[e], wd_buf.at[sl], sems.at[sl]).start()

    def _wait(e, sl):
        pltpu.make_async_copy(wg_hbm.at[e], wg_buf.at[sl], sems.at[sl]).wait()
        pltpu.make_async_copy(wu_hbm.at[e], wu_buf.at[sl], sems.at[sl]).wait()
        pltpu.make_async_copy(wd_hbm.at[e], wd_buf.at[sl], sems.at[sl]).wait()

    is_first = (smap_ref[b, 2] != 0) & (smap_ref[b, 1] != 0)

    @pl.when(b == 0)
    def _():
        _start(smap_ref[b, 0], s)

    @pl.when(is_first)
    def _():
        _wait(smap_ref[b, 0], s)
        wg16[...] = wg_buf[s].astype(jnp.bfloat16)
        wu16[...] = wu_buf[s].astype(jnp.bfloat16)
        wd16[...] = wd_buf[s].astype(jnp.bfloat16)

    @pl.when(is_first & (smap_ref[b, 5] != 0))
    def _():
        _start(smap_ref[b, 4], 1 - s)

    @pl.when(smap_ref[b, 1] != 0)
    def _():
        xa, xb_ = _unpack(xs_ref[...])
        xa = xa.astype(jnp.bfloat16)
        xb_ = xb_.astype(jnp.bfloat16)
        g = (jnp.dot(xa, wg16[:DH, :], preferred_element_type=jnp.float32)
             + jnp.dot(xb_, wg16[DH:, :], preferred_element_type=jnp.float32)
             + bg_ref[0])
        u = (jnp.dot(xa, wu16[:DH, :], preferred_element_type=jnp.float32)
             + jnp.dot(xb_, wu16[DH:, :], preferred_element_type=jnp.float32)
             + bu_ref[0])
        h = (g * (u * jax.nn.sigmoid(u))).astype(jnp.bfloat16)
        ya = (jnp.dot(h, wd16[:, :DH], preferred_element_type=jnp.float32)
              + bd_ref[0][:, :DH])
        yb = (jnp.dot(h, wd16[:, DH:], preferred_element_type=jnp.float32)
              + bd_ref[0][:, DH:])
        y_ref[...] = _pack(ya, yb)


_ffn_call = pl.pallas_call(
    _ffn_body,
    grid_spec=pltpu.PrefetchScalarGridSpec(
        num_scalar_prefetch=1,
        grid=(NB,),
        in_specs=[
            pl.BlockSpec((BR, DH), lambda b, sm: (b, 0)),
            pl.BlockSpec(memory_space=pl.ANY),
            pl.BlockSpec(memory_space=pl.ANY),
            pl.BlockSpec(memory_space=pl.ANY),
            pl.BlockSpec((1, 1, HD), lambda b, sm: (sm[b, 0], 0, 0)),
            pl.BlockSpec((1, 1, HD), lambda b, sm: (sm[b, 0], 0, 0)),
            pl.BlockSpec((1, 1, D), lambda b, sm: (sm[b, 0], 0, 0)),
        ],
        out_specs=pl.BlockSpec((BR, DH), lambda b, sm: (b, 0)),
        scratch_shapes=[
            pltpu.VMEM((2, D, HD), jnp.float32),
            pltpu.VMEM((2, D, HD), jnp.float32),
            pltpu.VMEM((2, HD, D), jnp.float32),
            pltpu.VMEM((D, HD), jnp.bfloat16),
            pltpu.VMEM((D, HD), jnp.bfloat16),
            pltpu.VMEM((HD, D), jnp.bfloat16),
            pltpu.SemaphoreType.DMA((2,)),
        ],
    ),
    out_shape=jax.ShapeDtypeStruct((PT, DH), jnp.int32),
)


def _combine_body(ye_ref, yo_ref, w_ref, out_ref):
    ea, eb = _unpack(ye_ref[...])
    oa, ob = _unpack(yo_ref[...])
    w0 = w_ref[:, 0:1]
    w1 = w_ref[:, 1:2]
    out_ref[0] = jnp.concatenate(
        [ea * w0 + oa * w1, eb * w0 + ob * w1], axis=1)


_combine_call = pl.pallas_call(
    _combine_body,
    grid=(N // 1024,),
    in_specs=[
        pl.BlockSpec((1024, DH), lambda b: (b, 0)),
        pl.BlockSpec((1024, DH), lambda b: (b, 0)),
        pl.BlockSpec((1024, K), lambda b: (b, 0)),
    ],
    out_specs=pl.BlockSpec((1, 1024, D), lambda b: (0, b, 0)),
    out_shape=jax.ShapeDtypeStruct((1, N, D), jnp.float32),
)


def kernel(x, Wr, br, Wg, bg, Wu, bu, Wd, bd):
    Bv, Sv, Dv = x.shape
    (w, x16, destT, smap, aux) = _router_call(x, Wr, br.reshape(1, E))
    dispatch, combine_gather = _sc_kernels()
    xs = dispatch(x16, destT)
    ys = _ffn_call(smap, xs, Wg, Wu, Wd, bg.reshape(E, 1, HD),
                   bu.reshape(E, 1, HD), bd.reshape(E, 1, D))
    ye, yo = combine_gather(ys, destT)
    out = _combine_call(ye, yo, w)
    return out, aux[0, 0]
